# baseline (speedup 1.0000x reference)
import functools

import jax
import jax.numpy as jnp
from jax import lax
from jax.experimental import pallas as pl
from jax.experimental.pallas import tpu as pltpu
from jax.experimental.pallas import tpu_sc as plsc

B = 16384
D = 64
V = 1000000
NC = 2
NS = 16
NW = NC * NS
NVREG = B // 16
STUB_LANE = 999936
OUTW = 2 * D

_SC_MESH = plsc.VectorSubcoreMesh(
    core_axis_name="c", subcore_axis_name="s", num_cores=NC, num_subcores=NS
)


@functools.partial(
    pl.kernel,
    out_type=(
        jax.ShapeDtypeStruct((B + 16, OUTW), jnp.float32),
        jax.ShapeDtypeStruct((B + 16, OUTW), jnp.float32),
    ),
    mesh=_SC_MESH,
    scratch_types=[
        pltpu.VMEM((B,), jnp.int32),
        pltpu.VMEM((B + 16,), jnp.int32),
        pltpu.VMEM((B + 16,), jnp.int32),
        pltpu.VMEM((B + 16,), jnp.int32),
        pltpu.VMEM((B + 16,), jnp.int32),
        pltpu.VMEM((2, D, 128), jnp.float32),
        pltpu.VMEM((D, D), jnp.float32),
        pltpu.VMEM((16, OUTW), jnp.float32),
        pltpu.VMEM((1, 16), jnp.int32),
        pltpu.SemaphoreType.DMA,
        pltpu.SemaphoreType.DMA,
    ],
    compiler_params=pltpu.CompilerParams(needs_layout_passes=False),
)
def _sc_sweep_gather(user1, movie1, uembT, membT, u_out, m_out,
                     idxbuf, locr, locp, clr, clp, chunkbuf, stubbuf,
                     staging, posref, sem, ssem):
    wid = lax.axis_index("s") * NC + lax.axis_index("c")
    n_cols = jnp.where(wid < 4, 245, 244)
    s_col = jnp.where(wid < 4, wid * 245, 980 + (wid - 4) * 244)
    lo = s_col * 128
    hi = jnp.where(wid == NW - 1, V, (s_col + n_cols) * 128)
    iota = lax.iota(jnp.int32, 16)

    def do_table(tbl, idx_hbm, out_hbm):
        def issue(c, phase):
            return pltpu.async_copy(
                tbl.at[:, pl.ds(pl.multiple_of((s_col + c) * 128, 128), 128)],
                chunkbuf.at[phase], sem)

        issue(0, 0)
        pltpu.sync_copy(idx_hbm, idxbuf)

        def part_body(i, n):
            for u in range(8):
                v = i * 8 + u
                r = idxbuf[pl.ds(v * 16, 16)]
                pos = v * 16 + iota
                msk = (r >= lo) & (r < hi)
                pfx = plsc.cumsum(msk.astype(jnp.int32))
                slots = n + pfx - 1
                plsc.store_scatter(locr, [slots], r, mask=msk)
                plsc.store_scatter(locp, [slots], pos, mask=msk)
                n = n + pfx[15]
            return n
        n = lax.fori_loop(0, NVREG // 8, part_body, 0)
        locr[pl.ds(n, 16)] = jnp.full((16,), V + 7, jnp.int32)
        locp[pl.ds(n, 16)] = jnp.full((16,), B, jnp.int32)
        nlv = (n + 15) // 16

        def extract(gather_vals, clo, m):
            clr[pl.ds(m, 16)] = jnp.full((16,), clo, jnp.int32)
            clp[pl.ds(m, 16)] = jnp.full((16,), B, jnp.int32)

            def group_body(g, carry):
                rv = clr[pl.ds(g * 16, 16)]
                pv = clp[pl.ds(g * 16, 16)]
                cols = rv - clo
                for d in range(D):
                    vals = gather_vals(d, cols)
                    plsc.store_scatter(
                        staging, [iota, jnp.full((16,), d, jnp.int32)], vals)
                posref[0, :] = pv
                pltpu.async_copy(staging, out_hbm.at[posref.at[0]],
                                 ssem).wait()
                return carry
            lax.fori_loop(0, (m + 15) // 16, group_body, 0)

        def compress(clo, chi):
            def comp_body(j, m):
                r = locr[pl.ds(j * 16, 16)]
                p = locp[pl.ds(j * 16, 16)]
                msk = (r >= clo) & (r < chi)
                pfx = plsc.cumsum(msk.astype(jnp.int32))
                slots = m + pfx - 1
                plsc.store_scatter(clr, [slots], r, mask=msk)
                plsc.store_scatter(clp, [slots], p, mask=msk)
                return m + pfx[15]
            return lax.fori_loop(0, nlv, comp_body, 0)

        def chunk_body(c, carry):
            phase = c % 2
            @pl.when(c + 1 < n_cols)
            def _():
                issue(c + 1, (c + 1) % 2)
            pltpu.make_async_copy(
                tbl.at[:, pl.ds(pl.multiple_of((s_col + c) * 128, 128), 128)],
                chunkbuf.at[phase], sem).wait()
            clo = (s_col + c) * 128
            m = compress(clo, clo + 128)

            def gather_vals(d, cols):
                return plsc.load_gather(
                    chunkbuf,
                    [jnp.full((16,), phase, jnp.int32),
                     jnp.full((16,), d, jnp.int32), cols])
            extract(gather_vals, clo, m)
            return carry
        lax.fori_loop(0, n_cols, chunk_body, 0)

        @pl.when(wid == NW - 1)
        def _():
            pltpu.async_copy(tbl.at[:, pl.ds(STUB_LANE, D)], stubbuf,
                             sem).wait()
            m = compress(STUB_LANE, V)

            def gather_stub(d, cols):
                return plsc.load_gather(
                    stubbuf, [jnp.full((16,), d, jnp.int32), cols])
            extract(gather_stub, STUB_LANE, m)

    do_table(uembT, user1, u_out)
    do_table(membT, movie1, m_out)


BLK = 2048


def _mlp_body(u_ref, m_ref, w1_ref, b1_ref, w2_ref, b2_ref, w3_ref, b3_ref,
              wo_ref, bo_ref, out_ref):
    dn = (((1,), (1,)), ((), ()))
    u = u_ref[:, :D]
    m = m_ref[:, :D]
    w1 = w1_ref[...]
    h = lax.dot_general(u, w1[:, :D], dn, preferred_element_type=jnp.float32)
    h = h + lax.dot_general(m, w1[:, D:], dn, preferred_element_type=jnp.float32)
    h = jnp.maximum(h + b1_ref[...], 0.0)
    h = lax.dot_general(h, w2_ref[...], dn, preferred_element_type=jnp.float32)
    h = jnp.maximum(h + b2_ref[...], 0.0)
    h = lax.dot_general(h, w3_ref[...], dn, preferred_element_type=jnp.float32)
    h = jnp.maximum(h + b3_ref[...], 0.0)
    out_ref[...] = jnp.sum(h * wo_ref[0, :], axis=1) + bo_ref[...]


def _full(shape):
    return pl.BlockSpec(shape, lambda i: tuple(0 for _ in shape))


_mlp = pl.pallas_call(
    _mlp_body,
    grid=(B // BLK,),
    in_specs=[
        pl.BlockSpec((BLK, OUTW), lambda i: (i, 0)),
        pl.BlockSpec((BLK, OUTW), lambda i: (i, 0)),
        _full((256, 2 * D)),
        _full((256,)),
        _full((128, 256)),
        _full((128,)),
        _full((64, 128)),
        _full((64,)),
        _full((1, 64)),
        _full((1,)),
    ],
    out_specs=pl.BlockSpec((BLK,), lambda i: (i,)),
    out_shape=jax.ShapeDtypeStruct((B,), jnp.float32),
)


def kernel(user, movie, user_emb, movie_emb, W1, b1, W2, b2, W3, b3, Wo, bo):
    u_rows, m_rows = _sc_sweep_gather(
        user.astype(jnp.int32), movie.astype(jnp.int32),
        user_emb.T, movie_emb.T)
    return _mlp(u_rows, m_rows, W1, b1, W2, b2, W3, b3, Wo, bo)
